# baseline (speedup 1.0000x reference)
import functools

import jax
import jax.numpy as jnp
from jax import lax
from jax.experimental import pallas as pl
from jax.experimental.pallas import tpu as pltpu
from jax.experimental.pallas import tpu_sc as plsc

_NUM_CLASSES = 21
_THRESHOLD = 0.5
_NEGPOS_RATIO = 3
_V0 = 0.1
_V1 = 0.2
_P = 8732
_P_PAD = 8960
_B = 32
_O = 10


def _smooth_l1(d):
  a = jnp.abs(d)
  return jnp.where(a < 1.0, 0.5 * d * d, a - 0.5)


def _mbox_kernel(conf_ref, loc_ref, priors_ref, targets_ref,
                 out_ref, masked_ref, kf_ref):
  b = pl.program_id(0)

  f32 = jnp.float32
  i32 = jnp.int32

  lane_p = lax.broadcasted_iota(i32, (1, _P_PAD), 1)
  pad = lane_p >= _P

  t = targets_ref[0]
  tx1 = t[:, 0:1]
  ty1 = t[:, 1:2]
  tx2 = t[:, 2:3]
  ty2 = t[:, 3:4]
  tlab = t[:, 4:5]

  pcx = priors_ref[0:1, :]
  pcy = priors_ref[1:2, :]
  pw = priors_ref[2:3, :]
  ph = priors_ref[3:4, :]
  px1 = pcx - pw * 0.5
  py1 = pcy - ph * 0.5
  px2 = pcx + pw * 0.5
  py2 = pcy + ph * 0.5

  iw = jnp.maximum(jnp.minimum(tx2, px2) - jnp.maximum(tx1, px1), 0.0)
  ih = jnp.maximum(jnp.minimum(ty2, py2) - jnp.maximum(ty1, py1), 0.0)
  inter = iw * ih
  area_t = (tx2 - tx1) * (ty2 - ty1)
  area_p = (px2 - px1) * (py2 - py1)
  iou = inter / (area_t + area_p - inter)
  iou = jnp.where(jnp.broadcast_to(pad, iou.shape), -1.0, iou)

  row10 = lax.broadcasted_iota(i32, (_O, _P_PAD), 0)
  lane10 = lax.broadcasted_iota(i32, (_O, _P_PAD), 1)

  bt_ov = jnp.max(iou, axis=0, keepdims=True)
  bt_idx = jnp.min(
      jnp.where(iou == bt_ov, row10, _O), axis=0, keepdims=True)

  bp_ov = jnp.max(iou, axis=1, keepdims=True)
  bp_idx = jnp.min(
      jnp.where(iou == bp_ov, lane10, _P_PAD), axis=1, keepdims=True)

  is_best = lane10 == bp_idx
  winner = jnp.max(jnp.where(is_best, row10, -1), axis=0, keepdims=True)
  bt_ov = jnp.where(winner >= 0, 2.0, bt_ov)
  bt_idx = jnp.where(winner >= 0, winner, bt_idx)

  onehot = (bt_idx == row10).astype(f32)
  mx1 = jnp.sum(onehot * tx1, axis=0, keepdims=True)
  my1 = jnp.sum(onehot * ty1, axis=0, keepdims=True)
  mx2 = jnp.sum(onehot * tx2, axis=0, keepdims=True)
  my2 = jnp.sum(onehot * ty2, axis=0, keepdims=True)
  mlab = jnp.sum(onehot * tlab, axis=0, keepdims=True)

  posm = bt_ov >= _THRESHOLD
  conf_t = jnp.where(posm, mlab + 1.0, 0.0)
  posf = posm.astype(f32)

  g_cx = ((mx1 + mx2) * 0.5 - pcx) / (_V0 * pw)
  g_cy = ((my1 + my2) * 0.5 - pcy) / (_V0 * ph)
  g_w = jnp.log((mx2 - mx1) / pw) / _V1
  g_h = jnp.log((my2 - my1) / ph) / _V1

  l = loc_ref[0]
  sl1 = (_smooth_l1(l[0:1, :] - g_cx) + _smooth_l1(l[1:2, :] - g_cy) +
         _smooth_l1(l[2:3, :] - g_w) + _smooth_l1(l[3:4, :] - g_h))
  loss_l_b = jnp.sum(sl1 * posf)

  conf = conf_ref[0]
  cmax = jnp.max(conf, axis=0, keepdims=True)
  ssum = jnp.sum(jnp.exp(conf - cmax), axis=0, keepdims=True)
  lse = jnp.log(ssum) + cmax

  cls_iota = lax.broadcasted_iota(i32, (_NUM_CLASSES, _P_PAD), 0)
  conf_t_i = conf_t.astype(i32)
  gathered = jnp.sum(
      jnp.where(cls_iota == conf_t_i, conf, 0.0), axis=0, keepdims=True)
  ce = lse - gathered

  ce_pos_sum = jnp.sum(jnp.where(posm, ce, 0.0))
  num_pos_f = jnp.sum(posf)
  k_f = jnp.minimum(_NEGPOS_RATIO * num_pos_f, float(_P - 1))

  masked = jnp.maximum(jnp.where(posm | pad, 0.0, ce), 0.0)
  masked_ref[0] = masked
  kf_ref[0] = jnp.broadcast_to(k_f, (1, 128))

  lane_o = lax.broadcasted_iota(i32, (8, 128), 1)
  contrib = jnp.where(
      lane_o == 0, loss_l_b,
      jnp.where(lane_o == 1, ce_pos_sum,
                jnp.where(lane_o == 2, num_pos_f, 0.0)))

  @pl.when(b == 0)
  def _init():
    out_ref[...] = contrib

  @pl.when(b > 0)
  def _acc():
    out_ref[...] += contrib


def _sc_mine_body(masked_hbm, kf_hbm, out_hbm, mrow, ibits, kfrow, outv,
                  nc):
  f32 = jnp.float32
  i32 = jnp.int32
  wid = lax.axis_index("s") * nc + lax.axis_index("c")

  pltpu.sync_copy(masked_hbm.at[wid], mrow)
  pltpu.sync_copy(kf_hbm.at[wid], kfrow)
  k = jnp.max(kfrow[pl.ds(0, 16)])

  n_outer = _P_PAD // 128

  def pre_body(j, hm):
    base = pl.multiple_of(j * 128, 128)
    for u in range(8):
      v = mrow[pl.ds(base + u * 16, 16)]
      ib = plsc.bitcast(v, i32)
      ibits[pl.ds(base + u * 16, 16)] = ib
      hm = jnp.maximum(hm, ib)
    return hm

  hmax = lax.fori_loop(0, n_outer, pre_body, jnp.zeros((16,), i32))
  hi0 = jnp.max(hmax)
  lo0 = jnp.zeros((), i32)

  def bs_body(_, carry):
    lo, hi = carry
    mid = lo + lax.shift_right_logical(hi - lo + 1, 1)

    def cnt_body(j, acc):
      base = pl.multiple_of(j * 128, 128)
      for u in range(8):
        ib = ibits[pl.ds(base + u * 16, 16)]
        acc = acc + jnp.where(ib >= mid, 1.0, 0.0)
      return acc

    acc = lax.fori_loop(0, n_outer, cnt_body, jnp.zeros((16,), f32))
    take = jnp.sum(acc) >= k
    return (jnp.where(take, mid, lo), jnp.where(take, hi, mid - 1))

  lo_fin, _ = lax.fori_loop(0, 31, bs_body, (lo0, hi0))

  def fin_body(j, carry):
    sg, cg, tv = carry
    base = pl.multiple_of(j * 128, 128)
    for u in range(8):
      ib = ibits[pl.ds(base + u * 16, 16)]
      v = mrow[pl.ds(base + u * 16, 16)]
      gt = ib > lo_fin
      sg = sg + jnp.where(gt, v, 0.0)
      cg = cg + jnp.where(gt, 1.0, 0.0)
      tv = jnp.maximum(tv, jnp.where(gt, 0.0, v))
    return sg, cg, tv

  z = jnp.zeros((16,), f32)
  sg, cg, tv = lax.fori_loop(0, n_outer, fin_body, (z, z, z))
  topk = jnp.sum(sg) + (k - jnp.sum(cg)) * jnp.max(tv)

  outv[...] = jnp.broadcast_to(topk, (16,))
  pltpu.sync_copy(outv, out_hbm.at[wid])


@jax.jit
def kernel(loc_data, conf_data, priors, targets):
  batch = loc_data.shape[0]
  pad_n = _P_PAD - _P

  conf_t_in = jnp.pad(jnp.transpose(conf_data, (0, 2, 1)),
                      ((0, 0), (0, 0), (0, pad_n)))
  loc_t_in = jnp.pad(jnp.transpose(loc_data, (0, 2, 1)),
                     ((0, 0), (0, 0), (0, pad_n)))
  pad_priors = jnp.broadcast_to(
      jnp.array([[10.0, 10.0, 0.1, 0.1]], jnp.float32), (pad_n, 4))
  priors_in = jnp.transpose(jnp.concatenate([priors, pad_priors], axis=0))

  stats, masked3d, kf3d = pl.pallas_call(
      _mbox_kernel,
      grid=(batch,),
      in_specs=[
          pl.BlockSpec((1, _NUM_CLASSES, _P_PAD), lambda b: (b, 0, 0)),
          pl.BlockSpec((1, 4, _P_PAD), lambda b: (b, 0, 0)),
          pl.BlockSpec((4, _P_PAD), lambda b: (0, 0)),
          pl.BlockSpec((1, _O, 5), lambda b: (b, 0, 0)),
      ],
      out_specs=[
          pl.BlockSpec((8, 128), lambda b: (0, 0)),
          pl.BlockSpec((1, 1, _P_PAD), lambda b: (b, 0, 0)),
          pl.BlockSpec((1, 1, 128), lambda b: (b, 0, 0)),
      ],
      out_shape=[
          jax.ShapeDtypeStruct((8, 128), jnp.float32),
          jax.ShapeDtypeStruct((_B, 1, _P_PAD), jnp.float32),
          jax.ShapeDtypeStruct((_B, 1, 128), jnp.float32),
      ],
      compiler_params=pltpu.CompilerParams(
          dimension_semantics=("arbitrary",)),
  )(conf_t_in, loc_t_in, priors_in, targets)

  info = plsc.get_sparse_core_info()
  nc = info.num_cores
  mesh = plsc.VectorSubcoreMesh(core_axis_name="c", subcore_axis_name="s")

  sc_mine = functools.partial(
      pl.kernel,
      out_type=jax.ShapeDtypeStruct((_B, 16), jnp.float32),
      mesh=mesh,
      scratch_types=[
          pltpu.VMEM((_P_PAD,), jnp.float32),
          pltpu.VMEM((_P_PAD,), jnp.int32),
          pltpu.VMEM((128,), jnp.float32),
          pltpu.VMEM((16,), jnp.float32),
      ],
      compiler_params=pltpu.CompilerParams(needs_layout_passes=False),
  )(functools.partial(_sc_mine_body, nc=nc))

  topk_rows = sc_mine(masked3d.reshape(_B, _P_PAD),
                      kf3d.reshape(_B, 128))

  loss_l = stats[0, 0]
  loss_c = stats[0, 1] + jnp.sum(topk_rows[:, 0])
  n = stats[0, 2]
  n = jnp.where(n == 0.0, jnp.float32(batch), n)
  return (loss_l / n, loss_c / n)


# speedup vs baseline: 1.2127x; 1.2127x over previous
import functools

import jax
import jax.numpy as jnp
from jax import lax
from jax.experimental import pallas as pl
from jax.experimental.pallas import tpu as pltpu
from jax.experimental.pallas import tpu_sc as plsc

_NUM_CLASSES = 21
_THRESHOLD = 0.5
_NEGPOS_RATIO = 3
_V0 = 0.1
_V1 = 0.2
_P = 8732
_P_PAD = 8960
_B = 32
_O = 10


def _smooth_l1(d):
  a = jnp.abs(d)
  return jnp.where(a < 1.0, 0.5 * d * d, a - 0.5)


def _mbox_kernel(conf_ref, loc_ref, priors_ref, targets_ref,
                 out_ref, masked_ref, kf_ref):
  b = pl.program_id(0)

  f32 = jnp.float32
  i32 = jnp.int32

  lane_p = lax.broadcasted_iota(i32, (1, _P_PAD), 1)
  pad = lane_p >= _P

  t = targets_ref[0]
  tx1 = t[:, 0:1]
  ty1 = t[:, 1:2]
  tx2 = t[:, 2:3]
  ty2 = t[:, 3:4]
  tlab = t[:, 4:5]

  pcx = priors_ref[0:1, :]
  pcy = priors_ref[1:2, :]
  pw = priors_ref[2:3, :]
  ph = priors_ref[3:4, :]
  px1 = pcx - pw * 0.5
  py1 = pcy - ph * 0.5
  px2 = pcx + pw * 0.5
  py2 = pcy + ph * 0.5

  iw = jnp.maximum(jnp.minimum(tx2, px2) - jnp.maximum(tx1, px1), 0.0)
  ih = jnp.maximum(jnp.minimum(ty2, py2) - jnp.maximum(ty1, py1), 0.0)
  inter = iw * ih
  area_t = (tx2 - tx1) * (ty2 - ty1)
  area_p = (px2 - px1) * (py2 - py1)
  iou = inter / (area_t + area_p - inter)
  iou = jnp.where(jnp.broadcast_to(pad, iou.shape), -1.0, iou)

  row10 = lax.broadcasted_iota(i32, (_O, _P_PAD), 0)
  lane10 = lax.broadcasted_iota(i32, (_O, _P_PAD), 1)

  bt_ov = jnp.max(iou, axis=0, keepdims=True)
  bt_idx = jnp.min(
      jnp.where(iou == bt_ov, row10, _O), axis=0, keepdims=True)

  bp_ov = jnp.max(iou, axis=1, keepdims=True)
  bp_idx = jnp.min(
      jnp.where(iou == bp_ov, lane10, _P_PAD), axis=1, keepdims=True)

  is_best = lane10 == bp_idx
  winner = jnp.max(jnp.where(is_best, row10, -1), axis=0, keepdims=True)
  bt_ov = jnp.where(winner >= 0, 2.0, bt_ov)
  bt_idx = jnp.where(winner >= 0, winner, bt_idx)

  onehot = (bt_idx == row10).astype(f32)
  matched = lax.dot_general(t, onehot, (((0,), (0,)), ((), ())),
                            preferred_element_type=f32)
  mx1 = matched[0:1, :]
  my1 = matched[1:2, :]
  mx2 = matched[2:3, :]
  my2 = matched[3:4, :]
  mlab = jnp.floor(matched[4:5, :] + 0.5)

  posm = bt_ov >= _THRESHOLD
  conf_t = jnp.where(posm, mlab + 1.0, 0.0)
  posf = posm.astype(f32)

  g_cx = ((mx1 + mx2) * 0.5 - pcx) / (_V0 * pw)
  g_cy = ((my1 + my2) * 0.5 - pcy) / (_V0 * ph)
  g_w = jnp.log((mx2 - mx1) / pw) / _V1
  g_h = jnp.log((my2 - my1) / ph) / _V1

  l = loc_ref[0]
  sl1 = (_smooth_l1(l[0:1, :] - g_cx) + _smooth_l1(l[1:2, :] - g_cy) +
         _smooth_l1(l[2:3, :] - g_w) + _smooth_l1(l[3:4, :] - g_h))
  loss_l_b = jnp.sum(sl1 * posf)

  conf = conf_ref[0]
  cmax = jnp.max(conf, axis=0, keepdims=True)
  expc = jnp.exp(conf - cmax)
  ones21 = jnp.ones((1, _NUM_CLASSES), f32)
  ssum = lax.dot_general(ones21, expc, (((1,), (0,)), ((), ())),
                         preferred_element_type=f32)
  lse = jnp.log(ssum) + cmax

  cls_iota = lax.broadcasted_iota(i32, (_NUM_CLASSES, _P_PAD), 0)
  conf_t_i = conf_t.astype(i32)
  gathered = jnp.sum(
      jnp.where(cls_iota == conf_t_i, conf, 0.0), axis=0, keepdims=True)
  ce = lse - gathered

  ce_pos_sum = jnp.sum(jnp.where(posm, ce, 0.0))
  num_pos_f = jnp.sum(posf)
  k_f = jnp.minimum(_NEGPOS_RATIO * num_pos_f, float(_P - 1))

  masked = jnp.maximum(jnp.where(posm | pad, 0.0, ce), 0.0)
  masked_ref[0] = masked
  kf_ref[0] = jnp.broadcast_to(k_f, (1, 128))

  lane_o = lax.broadcasted_iota(i32, (8, 128), 1)
  contrib = jnp.where(
      lane_o == 0, loss_l_b,
      jnp.where(lane_o == 1, ce_pos_sum,
                jnp.where(lane_o == 2, num_pos_f, 0.0)))

  @pl.when(b == 0)
  def _init():
    out_ref[...] = contrib

  @pl.when(b > 0)
  def _acc():
    out_ref[...] += contrib


def _sc_mine_body(masked_hbm, kf_hbm, out_hbm, mrow, ibits, kfrow, outv,
                  nc):
  f32 = jnp.float32
  i32 = jnp.int32
  wid = lax.axis_index("s") * nc + lax.axis_index("c")

  pltpu.sync_copy(masked_hbm.at[wid], mrow)
  pltpu.sync_copy(kf_hbm.at[wid], kfrow)
  k = jnp.max(kfrow[pl.ds(0, 16)])

  n_outer = _P_PAD // 128

  def pre_body(j, hm):
    base = pl.multiple_of(j * 128, 128)
    for u in range(8):
      v = mrow[pl.ds(base + u * 16, 16)]
      ib = plsc.bitcast(v, i32)
      ibits[pl.ds(base + u * 16, 16)] = ib
      hm = jnp.maximum(hm, ib)
    return hm

  hmax = lax.fori_loop(0, n_outer, pre_body, jnp.zeros((16,), i32))
  hi0 = jnp.max(hmax)
  lo0 = jnp.zeros((), i32)

  def bs_body(_, carry):
    lo, hi = carry
    mid = lo + lax.shift_right_logical(hi - lo + 1, 1)

    def cnt_body(j, acc):
      base = pl.multiple_of(j * 128, 128)
      for u in range(8):
        ib = ibits[pl.ds(base + u * 16, 16)]
        acc = acc + jnp.where(ib >= mid, 1.0, 0.0)
      return acc

    acc = lax.fori_loop(0, n_outer, cnt_body, jnp.zeros((16,), f32))
    take = jnp.sum(acc) >= k
    return (jnp.where(take, mid, lo), jnp.where(take, hi, mid - 1))

  lo_fin, _ = lax.fori_loop(0, 31, bs_body, (lo0, hi0))

  def fin_body(j, carry):
    sg, cg, tv = carry
    base = pl.multiple_of(j * 128, 128)
    for u in range(8):
      ib = ibits[pl.ds(base + u * 16, 16)]
      v = mrow[pl.ds(base + u * 16, 16)]
      gt = ib > lo_fin
      sg = sg + jnp.where(gt, v, 0.0)
      cg = cg + jnp.where(gt, 1.0, 0.0)
      tv = jnp.maximum(tv, jnp.where(gt, 0.0, v))
    return sg, cg, tv

  z = jnp.zeros((16,), f32)
  sg, cg, tv = lax.fori_loop(0, n_outer, fin_body, (z, z, z))
  topk = jnp.sum(sg) + (k - jnp.sum(cg)) * jnp.max(tv)

  outv[...] = jnp.broadcast_to(topk, (16,))
  pltpu.sync_copy(outv, out_hbm.at[wid])


@jax.jit
def kernel(loc_data, conf_data, priors, targets):
  batch = loc_data.shape[0]
  pad_n = _P_PAD - _P

  conf_t_in = jnp.pad(jnp.transpose(conf_data, (0, 2, 1)),
                      ((0, 0), (0, 0), (0, pad_n)))
  loc_t_in = jnp.pad(jnp.transpose(loc_data, (0, 2, 1)),
                     ((0, 0), (0, 0), (0, pad_n)))
  pad_priors = jnp.broadcast_to(
      jnp.array([[10.0, 10.0, 0.1, 0.1]], jnp.float32), (pad_n, 4))
  priors_in = jnp.transpose(jnp.concatenate([priors, pad_priors], axis=0))

  stats, masked3d, kf3d = pl.pallas_call(
      _mbox_kernel,
      grid=(batch,),
      in_specs=[
          pl.BlockSpec((1, _NUM_CLASSES, _P_PAD), lambda b: (b, 0, 0)),
          pl.BlockSpec((1, 4, _P_PAD), lambda b: (b, 0, 0)),
          pl.BlockSpec((4, _P_PAD), lambda b: (0, 0)),
          pl.BlockSpec((1, _O, 5), lambda b: (b, 0, 0)),
      ],
      out_specs=[
          pl.BlockSpec((8, 128), lambda b: (0, 0)),
          pl.BlockSpec((1, 1, _P_PAD), lambda b: (b, 0, 0)),
          pl.BlockSpec((1, 1, 128), lambda b: (b, 0, 0)),
      ],
      out_shape=[
          jax.ShapeDtypeStruct((8, 128), jnp.float32),
          jax.ShapeDtypeStruct((_B, 1, _P_PAD), jnp.float32),
          jax.ShapeDtypeStruct((_B, 1, 128), jnp.float32),
      ],
      compiler_params=pltpu.CompilerParams(
          dimension_semantics=("arbitrary",)),
  )(conf_t_in, loc_t_in, priors_in, targets)

  info = plsc.get_sparse_core_info()
  nc = info.num_cores
  mesh = plsc.VectorSubcoreMesh(core_axis_name="c", subcore_axis_name="s")

  sc_mine = functools.partial(
      pl.kernel,
      out_type=jax.ShapeDtypeStruct((_B, 16), jnp.float32),
      mesh=mesh,
      scratch_types=[
          pltpu.VMEM((_P_PAD,), jnp.float32),
          pltpu.VMEM((_P_PAD,), jnp.int32),
          pltpu.VMEM((128,), jnp.float32),
          pltpu.VMEM((16,), jnp.float32),
      ],
      compiler_params=pltpu.CompilerParams(needs_layout_passes=False),
  )(functools.partial(_sc_mine_body, nc=nc))

  topk_rows = sc_mine(masked3d.reshape(_B, _P_PAD),
                      kf3d.reshape(_B, 128))

  loss_l = stats[0, 0]
  loss_c = stats[0, 1] + jnp.sum(topk_rows[:, 0])
  n = stats[0, 2]
  n = jnp.where(n == 0.0, jnp.float32(batch), n)
  return (loss_l / n, loss_c / n)


# speedup vs baseline: 1.2133x; 1.0005x over previous
import functools

import jax
import jax.numpy as jnp
from jax import lax
from jax.experimental import pallas as pl
from jax.experimental.pallas import tpu as pltpu
from jax.experimental.pallas import tpu_sc as plsc

_NUM_CLASSES = 21
_THRESHOLD = 0.5
_NEGPOS_RATIO = 3
_V0 = 0.1
_V1 = 0.2
_P = 8732
_P_PAD = 8960
_B = 32
_O = 10


def _smooth_l1(d):
  a = jnp.abs(d)
  return jnp.where(a < 1.0, 0.5 * d * d, a - 0.5)


def _mbox_kernel(conf_ref, loc_ref, priors_ref, targets_ref,
                 out_ref, masked_ref, kf_ref):
  b = pl.program_id(0)

  f32 = jnp.float32
  i32 = jnp.int32

  lane_p = lax.broadcasted_iota(i32, (1, _P_PAD), 1)
  pad = lane_p >= _P

  t = targets_ref[0]
  tx1 = t[:, 0:1]
  ty1 = t[:, 1:2]
  tx2 = t[:, 2:3]
  ty2 = t[:, 3:4]
  tlab = t[:, 4:5]

  pcx = priors_ref[0:1, :]
  pcy = priors_ref[1:2, :]
  pw = priors_ref[2:3, :]
  ph = priors_ref[3:4, :]
  px1 = pcx - pw * 0.5
  py1 = pcy - ph * 0.5
  px2 = pcx + pw * 0.5
  py2 = pcy + ph * 0.5

  iw = jnp.maximum(jnp.minimum(tx2, px2) - jnp.maximum(tx1, px1), 0.0)
  ih = jnp.maximum(jnp.minimum(ty2, py2) - jnp.maximum(ty1, py1), 0.0)
  inter = iw * ih
  area_t = (tx2 - tx1) * (ty2 - ty1)
  area_p = (px2 - px1) * (py2 - py1)
  iou = inter / (area_t + area_p - inter)
  iou = jnp.where(jnp.broadcast_to(pad, iou.shape), -1.0, iou)

  row10 = lax.broadcasted_iota(i32, (_O, _P_PAD), 0)
  lane10 = lax.broadcasted_iota(i32, (_O, _P_PAD), 1)

  bt_ov = jnp.max(iou, axis=0, keepdims=True)
  bt_idx = jnp.min(
      jnp.where(iou == bt_ov, row10, _O), axis=0, keepdims=True)

  bp_ov = jnp.max(iou, axis=1, keepdims=True)
  bp_idx = jnp.min(
      jnp.where(iou == bp_ov, lane10, _P_PAD), axis=1, keepdims=True)

  is_best = lane10 == bp_idx
  winner = jnp.max(jnp.where(is_best, row10, -1), axis=0, keepdims=True)
  bt_ov = jnp.where(winner >= 0, 2.0, bt_ov)
  bt_idx = jnp.where(winner >= 0, winner, bt_idx)

  onehot = (bt_idx == row10).astype(f32)
  matched = lax.dot_general(t, onehot, (((0,), (0,)), ((), ())),
                            preferred_element_type=f32)
  mx1 = matched[0:1, :]
  my1 = matched[1:2, :]
  mx2 = matched[2:3, :]
  my2 = matched[3:4, :]
  mlab = jnp.floor(matched[4:5, :] + 0.5)

  posm = bt_ov >= _THRESHOLD
  conf_t = jnp.where(posm, mlab + 1.0, 0.0)
  posf = posm.astype(f32)

  g_cx = ((mx1 + mx2) * 0.5 - pcx) / (_V0 * pw)
  g_cy = ((my1 + my2) * 0.5 - pcy) / (_V0 * ph)
  g_w = jnp.log((mx2 - mx1) / pw) / _V1
  g_h = jnp.log((my2 - my1) / ph) / _V1

  l = loc_ref[0]
  sl1 = (_smooth_l1(l[0:1, :] - g_cx) + _smooth_l1(l[1:2, :] - g_cy) +
         _smooth_l1(l[2:3, :] - g_w) + _smooth_l1(l[3:4, :] - g_h))
  loss_l_b = jnp.sum(sl1 * posf)

  conf = conf_ref[0]
  cmax = jnp.max(conf, axis=0, keepdims=True)
  expc = jnp.exp(conf - cmax)
  ones21 = jnp.ones((1, _NUM_CLASSES), f32)
  ssum = lax.dot_general(ones21, expc, (((1,), (0,)), ((), ())),
                         preferred_element_type=f32)
  lse = jnp.log(ssum) + cmax

  cls_iota = lax.broadcasted_iota(i32, (_NUM_CLASSES, _P_PAD), 0)
  conf_t_i = conf_t.astype(i32)
  gathered = jnp.sum(
      jnp.where(cls_iota == conf_t_i, conf, 0.0), axis=0, keepdims=True)
  ce = lse - gathered

  ce_pos_sum = jnp.sum(jnp.where(posm, ce, 0.0))
  num_pos_f = jnp.sum(posf)
  k_f = jnp.minimum(_NEGPOS_RATIO * num_pos_f, float(_P - 1))

  masked = jnp.maximum(jnp.where(posm | pad, 0.0, ce), 0.0)
  masked_ref[0] = masked
  kf_ref[0] = jnp.broadcast_to(k_f, (1, 128))

  lane_o = lax.broadcasted_iota(i32, (8, 128), 1)
  contrib = jnp.where(
      lane_o == 0, loss_l_b,
      jnp.where(lane_o == 1, ce_pos_sum,
                jnp.where(lane_o == 2, num_pos_f, 0.0)))

  @pl.when(b == 0)
  def _init():
    out_ref[...] = contrib

  @pl.when(b > 0)
  def _acc():
    out_ref[...] += contrib


def _sc_mine_body(masked_hbm, kf_hbm, out_hbm, mrow, kfrow, outv, nc):
  f32 = jnp.float32
  i32 = jnp.int32
  wid = lax.axis_index("s") * nc + lax.axis_index("c")

  pltpu.sync_copy(masked_hbm.at[wid], mrow)
  pltpu.sync_copy(kf_hbm.at[wid], kfrow)
  k = jnp.max(kfrow[pl.ds(0, 16)])

  unroll = 16
  n_outer = _P_PAD // (16 * unroll)

  def pre_body(j, hm):
    base = pl.multiple_of(j * (16 * unroll), 16 * unroll)
    for u in range(unroll):
      hm = jnp.maximum(hm, mrow[pl.ds(base + u * 16, 16)])
    return hm

  hmax = lax.fori_loop(0, n_outer, pre_body, jnp.zeros((16,), f32))
  hi0 = jnp.max(plsc.bitcast(hmax, i32))
  lo0 = jnp.zeros((), i32)

  def bs_body(_, carry):
    lo, hi = carry
    mid = lo + lax.shift_right_logical(hi - lo + 1, 1)
    mid_f = plsc.bitcast(jnp.broadcast_to(mid, (16,)), f32)

    def cnt_body(j, acc):
      base = pl.multiple_of(j * (16 * unroll), 16 * unroll)
      for u in range(unroll):
        acc = acc + jnp.where(mrow[pl.ds(base + u * 16, 16)] >= mid_f,
                              1.0, 0.0)
      return acc

    acc = lax.fori_loop(0, n_outer, cnt_body, jnp.zeros((16,), f32))
    take = jnp.sum(acc) >= k
    return (jnp.where(take, mid, lo), jnp.where(take, hi, mid - 1))

  lo_fin, _ = lax.fori_loop(0, 31, bs_body, (lo0, hi0))
  lo_f = plsc.bitcast(jnp.broadcast_to(lo_fin, (16,)), f32)

  def fin_body(j, carry):
    sg, cg, tv = carry
    base = pl.multiple_of(j * (16 * unroll), 16 * unroll)
    for u in range(unroll):
      v = mrow[pl.ds(base + u * 16, 16)]
      gt = v > lo_f
      sg = sg + jnp.where(gt, v, 0.0)
      cg = cg + jnp.where(gt, 1.0, 0.0)
      tv = jnp.maximum(tv, jnp.where(gt, 0.0, v))
    return sg, cg, tv

  z = jnp.zeros((16,), f32)
  sg, cg, tv = lax.fori_loop(0, n_outer, fin_body, (z, z, z))
  topk = jnp.sum(sg) + (k - jnp.sum(cg)) * jnp.max(tv)

  outv[...] = jnp.broadcast_to(topk, (16,))
  pltpu.sync_copy(outv, out_hbm.at[wid])


@jax.jit
def kernel(loc_data, conf_data, priors, targets):
  batch = loc_data.shape[0]
  pad_n = _P_PAD - _P

  conf_t_in = jnp.pad(jnp.transpose(conf_data, (0, 2, 1)),
                      ((0, 0), (0, 0), (0, pad_n)))
  loc_t_in = jnp.pad(jnp.transpose(loc_data, (0, 2, 1)),
                     ((0, 0), (0, 0), (0, pad_n)))
  pad_priors = jnp.broadcast_to(
      jnp.array([[10.0, 10.0, 0.1, 0.1]], jnp.float32), (pad_n, 4))
  priors_in = jnp.transpose(jnp.concatenate([priors, pad_priors], axis=0))

  stats, masked3d, kf3d = pl.pallas_call(
      _mbox_kernel,
      grid=(batch,),
      in_specs=[
          pl.BlockSpec((1, _NUM_CLASSES, _P_PAD), lambda b: (b, 0, 0)),
          pl.BlockSpec((1, 4, _P_PAD), lambda b: (b, 0, 0)),
          pl.BlockSpec((4, _P_PAD), lambda b: (0, 0)),
          pl.BlockSpec((1, _O, 5), lambda b: (b, 0, 0)),
      ],
      out_specs=[
          pl.BlockSpec((8, 128), lambda b: (0, 0)),
          pl.BlockSpec((1, 1, _P_PAD), lambda b: (b, 0, 0)),
          pl.BlockSpec((1, 1, 128), lambda b: (b, 0, 0)),
      ],
      out_shape=[
          jax.ShapeDtypeStruct((8, 128), jnp.float32),
          jax.ShapeDtypeStruct((_B, 1, _P_PAD), jnp.float32),
          jax.ShapeDtypeStruct((_B, 1, 128), jnp.float32),
      ],
      compiler_params=pltpu.CompilerParams(
          dimension_semantics=("arbitrary",)),
  )(conf_t_in, loc_t_in, priors_in, targets)

  info = plsc.get_sparse_core_info()
  nc = info.num_cores
  mesh = plsc.VectorSubcoreMesh(core_axis_name="c", subcore_axis_name="s")

  sc_mine = functools.partial(
      pl.kernel,
      out_type=jax.ShapeDtypeStruct((_B, 16), jnp.float32),
      mesh=mesh,
      scratch_types=[
          pltpu.VMEM((_P_PAD,), jnp.float32),
          pltpu.VMEM((128,), jnp.float32),
          pltpu.VMEM((16,), jnp.float32),
      ],
      compiler_params=pltpu.CompilerParams(needs_layout_passes=False),
  )(functools.partial(_sc_mine_body, nc=nc))

  topk_rows = sc_mine(masked3d.reshape(_B, _P_PAD),
                      kf3d.reshape(_B, 128))

  loss_l = stats[0, 0]
  loss_c = stats[0, 1] + jnp.sum(topk_rows[:, 0])
  n = stats[0, 2]
  n = jnp.where(n == 0.0, jnp.float32(batch), n)
  return (loss_l / n, loss_c / n)
